# baseline (speedup 1.0000x reference)
import functools

import jax
import jax.numpy as jnp
from jax import lax
from jax.experimental import pallas as pl
from jax.experimental.pallas import tpu as pltpu
from jax.experimental.pallas import tpu_sc as plsc

_L = 16
_UNROLL = 8
_CHUNK = 12800


def _sc_convert(species, conv_tensor):
    n = species.shape[0] * species.shape[1]
    info = plsc.get_sparse_core_info()
    nc, ns = info.num_cores, info.num_subcores
    nw = nc * ns
    per_w = n // nw
    n_chunks = per_w // _CHUNK
    assert per_w * nw == n and n_chunks * _CHUNK == per_w and n_chunks % 2 == 0
    table_n = conv_tensor.shape[0]

    mesh = plsc.VectorSubcoreMesh(core_axis_name="c", subcore_axis_name="s")

    @functools.partial(
        pl.kernel,
        mesh=mesh,
        compiler_params=pltpu.CompilerParams(
            needs_layout_passes=False, use_tc_tiling_on_sc=False
        ),
        out_type=jax.ShapeDtypeStruct(species.shape, jnp.int32),
        scratch_types=[
            pltpu.VMEM((table_n,), jnp.int32),
            pltpu.VMEM((_CHUNK // species.shape[1], species.shape[1]), jnp.int32),
            pltpu.VMEM((_CHUNK // species.shape[1], species.shape[1]), jnp.int32),
            pltpu.VMEM((_CHUNK // species.shape[1], species.shape[1]), jnp.int32),
            pltpu.VMEM((_CHUNK // species.shape[1], species.shape[1]), jnp.int32),
            pltpu.SemaphoreType.DMA,
            pltpu.SemaphoreType.DMA,
            pltpu.SemaphoreType.DMA,
            pltpu.SemaphoreType.DMA,
        ],
    )
    def k(species_hbm, conv_hbm, out_hbm, table_v, in0, in1, out0, out1,
          s_in0, s_in1, s_out0, s_out1):
        wid = lax.axis_index("s") * nc + lax.axis_index("c")
        pltpu.sync_copy(conv_hbm, table_v)
        rows = species_hbm.shape[0]
        cols = species_hbm.shape[1]
        rows_per_chunk = _CHUNK // cols
        row0 = wid * (rows // nw)

        def in_slice(ci):
            return species_hbm.at[pl.ds(row0 + ci * rows_per_chunk, rows_per_chunk), :]

        def out_slice(ci):
            return out_hbm.at[pl.ds(row0 + ci * rows_per_chunk, rows_per_chunk), :]

        iota = lax.iota(jnp.int32, _L)
        assert cols == 200 and _CHUNK <= 43690

        def convert(in_2d, out_2d):
            def body(j, c):
                b = j * (_UNROLL * _L)
                for u in range(_UNROLL):
                    pos = iota + (b + u * _L)
                    r = lax.shift_right_logical(pos * 10486, 21)
                    col = pos - r * cols
                    sv = plsc.load_gather(in_2d, [r, col])
                    vals = plsc.load_gather(table_v, [sv])
                    plsc.store_scatter(out_2d, [r, col], vals)
                return c

            lax.fori_loop(0, _CHUNK // (_UNROLL * _L), body, 0)

        pltpu.async_copy(in_slice(0), in0, s_in0)
        pltpu.async_copy(in_slice(1), in1, s_in1)

        def round_body(i, c):
            c0 = 2 * i
            c1 = c0 + 1

            pltpu.make_async_copy(in_slice(c0), in0, s_in0).wait()

            @pl.when(i > 0)
            def _():
                pltpu.make_async_copy(out0, out_slice(c0), s_out0).wait()

            convert(in0, out0)
            pltpu.async_copy(out0, out_slice(c0), s_out0)

            @pl.when(c0 + 2 < n_chunks)
            def _():
                pltpu.async_copy(in_slice(c0 + 2), in0, s_in0)

            pltpu.make_async_copy(in_slice(c1), in1, s_in1).wait()

            @pl.when(i > 0)
            def _():
                pltpu.make_async_copy(out1, out_slice(c1), s_out1).wait()

            convert(in1, out1)
            pltpu.async_copy(out1, out_slice(c1), s_out1)

            @pl.when(c1 + 2 < n_chunks)
            def _():
                pltpu.async_copy(in_slice(c1 + 2), in1, s_in1)

            return c

        lax.fori_loop(0, n_chunks // 2, round_body, 0)

        pltpu.make_async_copy(out0, out_slice(n_chunks - 2), s_out0).wait()
        pltpu.make_async_copy(out1, out_slice(n_chunks - 1), s_out1).wait()

    return k(species, conv_tensor)


def kernel(species, coordinates, conv_tensor):
    return _sc_convert(species, conv_tensor), coordinates


# speedup vs baseline: 1.3677x; 1.3677x over previous
import functools

import jax
import jax.numpy as jnp
from jax import lax
from jax.experimental import pallas as pl
from jax.experimental.pallas import tpu as pltpu
from jax.experimental.pallas import tpu_sc as plsc

_L = 16
_ROWS = 64


def _sc_convert(species, conv_tensor):
    n_rows, n_cols = species.shape
    info = plsc.get_sparse_core_info()
    nc, ns = info.num_cores, info.num_subcores
    nw = nc * ns
    rows_w = n_rows // nw
    n_chunks = rows_w // _ROWS
    assert rows_w * nw == n_rows and n_chunks * _ROWS == rows_w
    assert n_chunks % 2 == 0
    cols_a = 128
    cols_b = n_cols - cols_a
    assert 0 < cols_b <= 128
    b_full = cols_b // _L
    b_tail = cols_b - b_full * _L
    table_n = conv_tensor.shape[0]

    mesh = plsc.VectorSubcoreMesh(core_axis_name="c", subcore_axis_name="s")

    @functools.partial(
        pl.kernel,
        mesh=mesh,
        compiler_params=pltpu.CompilerParams(needs_layout_passes=False),
        out_type=jax.ShapeDtypeStruct(species.shape, jnp.int32),
        scratch_types=[
            pltpu.VMEM((table_n,), jnp.int32),
            pltpu.VMEM((2, _ROWS, cols_a), jnp.int32),
            pltpu.VMEM((2, _ROWS, cols_b), jnp.int32),
            pltpu.VMEM((2, _ROWS, cols_a), jnp.int32),
            pltpu.VMEM((2, _ROWS, cols_b), jnp.int32),
            pltpu.SemaphoreType.DMA,
            pltpu.SemaphoreType.DMA,
            pltpu.SemaphoreType.DMA,
            pltpu.SemaphoreType.DMA,
        ],
    )
    def k(species_hbm, conv_hbm, out_hbm, table_v, in_a, in_b, out_a, out_b,
          s_in0, s_in1, s_out0, s_out1):
        wid = lax.axis_index("s") * nc + lax.axis_index("c")
        pltpu.sync_copy(conv_hbm, table_v)
        row0 = wid * rows_w
        s_in = (s_in0, s_in1)
        s_out = (s_out0, s_out1)
        iota = lax.iota(jnp.int32, _L)
        tail_cols = cols_a + b_full * _L + iota
        tail_mask = iota < b_tail

        def hbm_a(ci):
            return species_hbm.at[pl.ds(row0 + ci * _ROWS, _ROWS), pl.ds(0, cols_a)]

        def hbm_b(ci):
            return species_hbm.at[
                pl.ds(row0 + ci * _ROWS, _ROWS), pl.ds(cols_a, cols_b)]

        def hbm_oa(ci):
            return out_hbm.at[pl.ds(row0 + ci * _ROWS, _ROWS), pl.ds(0, cols_a)]

        def hbm_ob(ci):
            return out_hbm.at[pl.ds(row0 + ci * _ROWS, _ROWS), pl.ds(cols_a, cols_b)]

        def start_in(ci, b):
            pltpu.async_copy(hbm_a(ci), in_a.at[b], s_in[b])
            pltpu.async_copy(hbm_b(ci), in_b.at[b], s_in[b])

        def wait_in(ci, b):
            pltpu.make_async_copy(hbm_a(ci), in_a.at[b], s_in[b]).wait()
            pltpu.make_async_copy(hbm_b(ci), in_b.at[b], s_in[b]).wait()

        def start_out(ci, b):
            pltpu.async_copy(out_a.at[b], hbm_oa(ci), s_out[b])
            pltpu.async_copy(out_b.at[b], hbm_ob(ci), s_out[b])

        def wait_out(ci, b):
            pltpu.make_async_copy(out_a.at[b], hbm_oa(ci), s_out[b]).wait()
            pltpu.make_async_copy(out_b.at[b], hbm_ob(ci), s_out[b]).wait()

        def convert(b):
            ia, ib = in_a.at[b], in_b.at[b]
            oa, ob = out_a.at[b], out_b.at[b]

            def row_body(r, c):
                for u in range(cols_a // _L):
                    idx = ia[r, pl.ds(u * _L, _L)]
                    oa[r, pl.ds(u * _L, _L)] = plsc.load_gather(table_v, [idx])
                for u in range(b_full):
                    idx = ib[r, pl.ds(u * _L, _L)]
                    ob[r, pl.ds(u * _L, _L)] = plsc.load_gather(table_v, [idx])
                if b_tail:
                    rr = jnp.full((_L,), r, dtype=jnp.int32)
                    cc = b_full * _L + iota
                    idx = plsc.load_gather(ib, [rr, cc], mask=tail_mask)
                    vals = plsc.load_gather(table_v, [idx], mask=tail_mask)
                    plsc.store_scatter(ob, [rr, cc], vals, mask=tail_mask)
                return c

            lax.fori_loop(0, _ROWS, row_body, 0)

        start_in(0, 0)
        start_in(1, 1)

        def round_body(i, c):
            c0 = 2 * i
            c1 = c0 + 1

            wait_in(c0, 0)

            @pl.when(i > 0)
            def _():
                wait_out(c0, 0)

            convert(0)
            start_out(c0, 0)

            @pl.when(c0 + 2 < n_chunks)
            def _():
                start_in(c0 + 2, 0)

            wait_in(c1, 1)

            @pl.when(i > 0)
            def _():
                wait_out(c1, 1)

            convert(1)
            start_out(c1, 1)

            @pl.when(c1 + 2 < n_chunks)
            def _():
                start_in(c1 + 2, 1)

            return c

        lax.fori_loop(0, n_chunks // 2, round_body, 0)

        wait_out(n_chunks - 2, 0)
        wait_out(n_chunks - 1, 1)

    return k(species, conv_tensor)


def kernel(species, coordinates, conv_tensor):
    return _sc_convert(species, conv_tensor), coordinates


# speedup vs baseline: 2.0722x; 1.5151x over previous
import functools

import jax
import jax.numpy as jnp
from jax import lax
from jax.experimental import pallas as pl
from jax.experimental.pallas import tpu as pltpu
from jax.experimental.pallas import tpu_sc as plsc

_L = 16
_ROWS = 64


def _sc_convert(species, conv_tensor):
    n_rows, n_cols = species.shape
    info = plsc.get_sparse_core_info()
    nc, ns = info.num_cores, info.num_subcores
    nw = nc * ns
    rows_w = n_rows // nw
    n_chunks = rows_w // _ROWS
    assert rows_w * nw == n_rows and n_chunks * _ROWS == rows_w
    assert n_chunks % 2 == 0
    offsets = list(range(0, n_cols - _L + 1, _L))
    if n_cols % _L:
        offsets.append(n_cols - _L)
    table_n = conv_tensor.shape[0]

    mesh = plsc.VectorSubcoreMesh(core_axis_name="c", subcore_axis_name="s")

    @functools.partial(
        pl.kernel,
        mesh=mesh,
        compiler_params=pltpu.CompilerParams(needs_layout_passes=False),
        out_type=jax.ShapeDtypeStruct(species.shape, jnp.int32),
        scratch_types=[
            pltpu.VMEM((table_n,), jnp.int32),
            pltpu.VMEM((2, _ROWS, n_cols), jnp.int32),
            pltpu.VMEM((2, _ROWS, n_cols), jnp.int32),
            pltpu.SemaphoreType.DMA,
            pltpu.SemaphoreType.DMA,
            pltpu.SemaphoreType.DMA,
            pltpu.SemaphoreType.DMA,
        ],
    )
    def k(species_hbm, conv_hbm, out_hbm, table_v, in_v, out_v,
          s_in0, s_in1, s_out0, s_out1):
        wid = lax.axis_index("s") * nc + lax.axis_index("c")
        pltpu.sync_copy(conv_hbm, table_v)
        row0 = wid * rows_w
        s_in = (s_in0, s_in1)
        s_out = (s_out0, s_out1)

        def hbm_rows(ref, ci):
            return ref.at[pl.ds(row0 + ci * _ROWS, _ROWS), :]

        def start_in(ci, b):
            pltpu.async_copy(hbm_rows(species_hbm, ci), in_v.at[b], s_in[b])

        def wait_in(ci, b):
            pltpu.make_async_copy(hbm_rows(species_hbm, ci), in_v.at[b],
                                  s_in[b]).wait()

        def start_out(ci, b):
            pltpu.async_copy(out_v.at[b], hbm_rows(out_hbm, ci), s_out[b])

        def wait_out(ci, b):
            pltpu.make_async_copy(out_v.at[b], hbm_rows(out_hbm, ci),
                                  s_out[b]).wait()

        def convert(b):
            iv = in_v.at[b]
            ov = out_v.at[b]

            def row_body(r, c):
                idxs = [iv[r, pl.ds(off, _L)] for off in offsets]
                vals = [plsc.load_gather(table_v, [idx]) for idx in idxs]
                for off, val in zip(offsets, vals):
                    ov[r, pl.ds(off, _L)] = val
                return c

            lax.fori_loop(0, _ROWS, row_body, 0)

        start_in(0, 0)
        start_in(1, 1)

        def round_body(i, c):
            c0 = 2 * i
            c1 = c0 + 1

            wait_in(c0, 0)

            @pl.when(i > 0)
            def _():
                wait_out(c0, 0)

            convert(0)
            start_out(c0, 0)

            @pl.when(c0 + 2 < n_chunks)
            def _():
                start_in(c0 + 2, 0)

            wait_in(c1, 1)

            @pl.when(i > 0)
            def _():
                wait_out(c1, 1)

            convert(1)
            start_out(c1, 1)

            @pl.when(c1 + 2 < n_chunks)
            def _():
                start_in(c1 + 2, 1)

            return c

        lax.fori_loop(0, n_chunks // 2, round_body, 0)

        wait_out(n_chunks - 2, 0)
        wait_out(n_chunks - 1, 1)

    return k(species, conv_tensor)


def kernel(species, coordinates, conv_tensor):
    return _sc_convert(species, conv_tensor), coordinates


# speedup vs baseline: 3.0182x; 1.4565x over previous
import functools

import jax
import jax.numpy as jnp
from jax import lax
from jax.experimental import pallas as pl
from jax.experimental.pallas import tpu as pltpu
from jax.experimental.pallas import tpu_sc as plsc

_L = 16
_CW = 128
_RU = 2


def _sc_convert_t(species_t, conv_tensor):
    n_rows, n_cols = species_t.shape
    info = plsc.get_sparse_core_info()
    nc, ns = info.num_cores, info.num_subcores
    nw = nc * ns
    cols_w = n_cols // nw
    n_chunks = cols_w // _CW
    assert cols_w * nw == n_cols and n_chunks * _CW == cols_w
    assert n_chunks % 2 == 0 and n_rows % _RU == 0
    table_n = conv_tensor.shape[0]

    mesh = plsc.VectorSubcoreMesh(core_axis_name="c", subcore_axis_name="s")

    @functools.partial(
        pl.kernel,
        mesh=mesh,
        compiler_params=pltpu.CompilerParams(needs_layout_passes=False),
        out_type=jax.ShapeDtypeStruct(species_t.shape, jnp.int32),
        scratch_types=[
            pltpu.VMEM((table_n,), jnp.int32),
            pltpu.VMEM((2, n_rows, _CW), jnp.int32),
            pltpu.VMEM((2, n_rows, _CW), jnp.int32),
            pltpu.SemaphoreType.DMA,
            pltpu.SemaphoreType.DMA,
            pltpu.SemaphoreType.DMA,
            pltpu.SemaphoreType.DMA,
        ],
    )
    def k(species_hbm, conv_hbm, out_hbm, table_v, in_v, out_v,
          s_in0, s_in1, s_out0, s_out1):
        wid = lax.axis_index("s") * nc + lax.axis_index("c")
        pltpu.sync_copy(conv_hbm, table_v)
        col0 = wid * cols_w
        s_in = (s_in0, s_in1)
        s_out = (s_out0, s_out1)

        def hbm_cols(ref, ci):
            return ref.at[:, pl.ds(col0 + ci * _CW, _CW)]

        def start_in(ci, b):
            pltpu.async_copy(hbm_cols(species_hbm, ci), in_v.at[b], s_in[b])

        def wait_in(ci, b):
            pltpu.make_async_copy(hbm_cols(species_hbm, ci), in_v.at[b],
                                  s_in[b]).wait()

        def start_out(ci, b):
            pltpu.async_copy(out_v.at[b], hbm_cols(out_hbm, ci), s_out[b])

        def wait_out(ci, b):
            pltpu.make_async_copy(out_v.at[b], hbm_cols(out_hbm, ci),
                                  s_out[b]).wait()

        row_offsets = [(rr, off) for rr in range(_RU)
                       for off in range(0, _CW, _L)]

        def convert(b):
            iv = in_v.at[b]
            ov = out_v.at[b]

            def row_body(j, c):
                r = j * _RU
                idxs = [iv[r + rr, pl.ds(off, _L)] for rr, off in row_offsets]
                vals = [plsc.load_gather(table_v, [idx]) for idx in idxs]
                for (rr, off), val in zip(row_offsets, vals):
                    ov[r + rr, pl.ds(off, _L)] = val
                return c

            lax.fori_loop(0, n_rows // _RU, row_body, 0)

        start_in(0, 0)
        start_in(1, 1)

        def round_body(i, c):
            c0 = 2 * i
            c1 = c0 + 1

            wait_in(c0, 0)

            @pl.when(i > 0)
            def _():
                wait_out(c0, 0)

            convert(0)
            start_out(c0, 0)

            @pl.when(c0 + 2 < n_chunks)
            def _():
                start_in(c0 + 2, 0)

            wait_in(c1, 1)

            @pl.when(i > 0)
            def _():
                wait_out(c1, 1)

            convert(1)
            start_out(c1, 1)

            @pl.when(c1 + 2 < n_chunks)
            def _():
                start_in(c1 + 2, 1)

            return c

        lax.fori_loop(0, n_chunks // 2, round_body, 0)

        wait_out(n_chunks - 2, 0)
        wait_out(n_chunks - 1, 1)

    return k(species_t, conv_tensor)


def kernel(species, coordinates, conv_tensor):
    return _sc_convert_t(species.T, conv_tensor).T, coordinates


# speedup vs baseline: 3.3918x; 1.1238x over previous
import functools

import jax
import jax.numpy as jnp
from jax import lax
from jax.experimental import pallas as pl
from jax.experimental.pallas import tpu as pltpu
from jax.experimental.pallas import tpu_sc as plsc

_L = 16
_CW = 128
_RU = 2


def _sc_convert_t(species_t, conv_tensor):
    n_rows, n_cols = species_t.shape
    info = plsc.get_sparse_core_info()
    nc, ns = info.num_cores, info.num_subcores
    nw = nc * ns
    cols_w = n_cols // nw
    n_chunks = cols_w // _CW
    assert cols_w * nw == n_cols and n_chunks * _CW == cols_w
    assert n_chunks % 2 == 0 and n_rows % _RU == 0
    table_n = conv_tensor.shape[0]

    mesh = plsc.VectorSubcoreMesh(core_axis_name="c", subcore_axis_name="s")

    @functools.partial(
        pl.kernel,
        mesh=mesh,
        compiler_params=pltpu.CompilerParams(needs_layout_passes=False),
        out_type=jax.ShapeDtypeStruct(species_t.shape, jnp.int32),
        scratch_types=[
            pltpu.VMEM((table_n,), jnp.int32),
            pltpu.VMEM((2, n_rows, _CW), jnp.int32),
            pltpu.VMEM((2, n_rows, _CW), jnp.int32),
            pltpu.SemaphoreType.DMA,
            pltpu.SemaphoreType.DMA,
            pltpu.SemaphoreType.DMA,
            pltpu.SemaphoreType.DMA,
        ],
    )
    def k(species_hbm, conv_hbm, out_hbm, table_v, in_v, out_v,
          s_in0, s_in1, s_out0, s_out1):
        wid = lax.axis_index("s") * nc + lax.axis_index("c")
        pltpu.sync_copy(conv_hbm, table_v)
        col0 = wid * cols_w
        s_in = (s_in0, s_in1)
        s_out = (s_out0, s_out1)

        def hbm_cols(ref, ci):
            return ref.at[:, pl.ds(col0 + ci * _CW, _CW)]

        def start_in(ci, b):
            pltpu.async_copy(hbm_cols(species_hbm, ci), in_v.at[b], s_in[b])

        def wait_in(ci, b):
            pltpu.make_async_copy(hbm_cols(species_hbm, ci), in_v.at[b],
                                  s_in[b]).wait()

        def start_out(ci, b):
            pltpu.async_copy(out_v.at[b], hbm_cols(out_hbm, ci), s_out[b])

        def wait_out(ci, b):
            pltpu.make_async_copy(out_v.at[b], hbm_cols(out_hbm, ci),
                                  s_out[b]).wait()

        row_offsets = [(rr, off) for rr in range(_RU)
                       for off in range(0, _CW, _L)]

        def convert(b):
            iv = in_v.at[b]
            ov = out_v.at[b]

            def row_body(j, c):
                r = j * _RU
                idxs = [iv[r + rr, pl.ds(off, _L)] for rr, off in row_offsets]
                vals = [plsc.load_gather(table_v, [idx]) for idx in idxs]
                for (rr, off), val in zip(row_offsets, vals):
                    ov[r + rr, pl.ds(off, _L)] = val
                return c

            lax.fori_loop(0, n_rows // _RU, row_body, 0)

        start_in(0, 0)
        start_in(1, 1)

        def round_body(i, c):
            c0 = 2 * i
            c1 = c0 + 1

            wait_in(c0, 0)

            @pl.when(i > 0)
            def _():
                wait_out(c0, 0)

            convert(0)
            start_out(c0, 0)

            @pl.when(c0 + 2 < n_chunks)
            def _():
                start_in(c0 + 2, 0)

            wait_in(c1, 1)

            @pl.when(i > 0)
            def _():
                wait_out(c1, 1)

            convert(1)
            start_out(c1, 1)

            @pl.when(c1 + 2 < n_chunks)
            def _():
                start_in(c1 + 2, 1)

            return c

        lax.fori_loop(0, n_chunks // 2, round_body, 0)

        wait_out(n_chunks - 2, 0)
        wait_out(n_chunks - 1, 1)

    return k(species_t, conv_tensor)


def _tc_copy_t(x_t):
    d0, d1, d2 = x_t.shape
    bc = 2048
    grid = (d0, d2 // bc)

    def body(x_ref, o_ref):
        o_ref[...] = x_ref[...]

    return pl.pallas_call(
        body,
        grid=grid,
        in_specs=[pl.BlockSpec((1, d1, bc), lambda i, j: (i, 0, j))],
        out_specs=pl.BlockSpec((1, d1, bc), lambda i, j: (i, 0, j)),
        out_shape=jax.ShapeDtypeStruct(x_t.shape, x_t.dtype),
    )(x_t)


def kernel(species, coordinates, conv_tensor):
    converted = _sc_convert_t(species.T, conv_tensor).T
    coords_out = _tc_copy_t(coordinates.transpose(2, 1, 0)).transpose(2, 1, 0)
    return converted, coords_out


# speedup vs baseline: 3.5520x; 1.0472x over previous
import functools

import jax
import jax.numpy as jnp
from jax import lax
from jax.experimental import pallas as pl
from jax.experimental.pallas import tpu as pltpu
from jax.experimental.pallas import tpu_sc as plsc

_L = 16
_CW = 128
_RU = 2


def _sc_convert_t(species_t, conv_tensor):
    n_rows, n_cols = species_t.shape
    info = plsc.get_sparse_core_info()
    nc, ns = info.num_cores, info.num_subcores
    nw = nc * ns
    cols_w = n_cols // nw
    n_chunks = cols_w // _CW
    assert cols_w * nw == n_cols and n_chunks * _CW == cols_w
    assert n_chunks % 2 == 0 and n_rows % _RU == 0
    table_n = conv_tensor.shape[0]

    mesh = plsc.VectorSubcoreMesh(core_axis_name="c", subcore_axis_name="s")

    @functools.partial(
        pl.kernel,
        mesh=mesh,
        compiler_params=pltpu.CompilerParams(needs_layout_passes=False),
        out_type=jax.ShapeDtypeStruct(species_t.shape, jnp.int32),
        scratch_types=[
            pltpu.VMEM((table_n,), jnp.int32),
            pltpu.VMEM((2, n_rows, _CW), jnp.int32),
            pltpu.VMEM((2, n_rows, _CW), jnp.int32),
            pltpu.SemaphoreType.DMA,
            pltpu.SemaphoreType.DMA,
            pltpu.SemaphoreType.DMA,
            pltpu.SemaphoreType.DMA,
        ],
    )
    def k(species_hbm, conv_hbm, out_hbm, table_v, in_v, out_v,
          s_in0, s_in1, s_out0, s_out1):
        wid = lax.axis_index("s") * nc + lax.axis_index("c")
        pltpu.sync_copy(conv_hbm, table_v)
        col0 = wid * cols_w
        s_in = (s_in0, s_in1)
        s_out = (s_out0, s_out1)

        def hbm_cols(ref, ci):
            return ref.at[:, pl.ds(col0 + ci * _CW, _CW)]

        def start_in(ci, b):
            pltpu.async_copy(hbm_cols(species_hbm, ci), in_v.at[b], s_in[b])

        def wait_in(ci, b):
            pltpu.make_async_copy(hbm_cols(species_hbm, ci), in_v.at[b],
                                  s_in[b]).wait()

        def start_out(ci, b):
            pltpu.async_copy(out_v.at[b], hbm_cols(out_hbm, ci), s_out[b])

        def wait_out(ci, b):
            pltpu.make_async_copy(out_v.at[b], hbm_cols(out_hbm, ci),
                                  s_out[b]).wait()

        row_offsets = [(rr, off) for rr in range(_RU)
                       for off in range(0, _CW, _L)]

        def convert(b):
            iv = in_v.at[b]
            ov = out_v.at[b]

            def row_body(j, c):
                r = j * _RU
                idxs = [iv[r + rr, pl.ds(off, _L)] for rr, off in row_offsets]
                vals = [plsc.load_gather(table_v, [idx]) for idx in idxs]
                for (rr, off), val in zip(row_offsets, vals):
                    ov[r + rr, pl.ds(off, _L)] = val
                return c

            lax.fori_loop(0, n_rows // _RU, row_body, 0)

        start_in(0, 0)
        start_in(1, 1)

        def round_body(i, c):
            c0 = 2 * i
            c1 = c0 + 1

            wait_in(c0, 0)

            @pl.when(i > 0)
            def _():
                wait_out(c0, 0)

            convert(0)
            start_out(c0, 0)

            @pl.when(c0 + 2 < n_chunks)
            def _():
                start_in(c0 + 2, 0)

            wait_in(c1, 1)

            @pl.when(i > 0)
            def _():
                wait_out(c1, 1)

            convert(1)
            start_out(c1, 1)

            @pl.when(c1 + 2 < n_chunks)
            def _():
                start_in(c1 + 2, 1)

            return c

        lax.fori_loop(0, n_chunks // 2, round_body, 0)

        wait_out(n_chunks - 2, 0)
        wait_out(n_chunks - 1, 1)

    return k(species_t, conv_tensor)


def _tc_copy_t(x_t):
    d0, d1, d2 = x_t.shape
    x2 = x_t.reshape(d0 * d1, d2)
    br = 40
    grid = (d0 * d1 // br,)

    def body(x_ref, o_ref):
        o_ref[...] = x_ref[...]

    out = pl.pallas_call(
        body,
        grid=grid,
        in_specs=[pl.BlockSpec((br, d2), lambda i: (i, 0))],
        out_specs=pl.BlockSpec((br, d2), lambda i: (i, 0)),
        out_shape=jax.ShapeDtypeStruct(x2.shape, x2.dtype),
    )(x2)
    return out.reshape(d0, d1, d2)


def kernel(species, coordinates, conv_tensor):
    converted = _sc_convert_t(species.T, conv_tensor).T
    coords_out = _tc_copy_t(coordinates.transpose(2, 1, 0)).transpose(2, 1, 0)
    return converted, coords_out
